# baseline (device time: 54558 ns/iter reference)
import os

import jax
import jax.numpy as jnp
from jax import lax
from jax.experimental import pallas as pl
from jax.experimental.pallas import tpu as pltpu

N_DEV = 4
ORDER = (2, 1, 3)
I16MAX = 32767.0
OUT_HBM = os.environ.get("OUT_HBM", "1") == "1"


def kernel(x, w_mat):
    m_per, k = x.shape
    _, n = w_mat.shape
    n_per = n // N_DEV

    def body(x_ref, w_ref, out_ref, ov_ref, w_buf, iout_ref, iin_ref,
             amax_send, amax_recv, w_sems, ov_sems, send_sems, recv_sems,
             am_send_sems, am_recv_sems):
        my = lax.axis_index("i")
        ov = ov_ref if OUT_HBM else out_ref

        barrier_sem = pltpu.get_barrier_semaphore()
        for d in range(1, N_DEV):
            pl.semaphore_signal(
                barrier_sem, inc=1,
                device_id=((my + d) % N_DEV,),
                device_id_type=pl.DeviceIdType.MESH,
            )
        pl.semaphore_wait(barrier_sem, N_DEV - 1)

        offs = list(ORDER) + [0]

        def w_fetch(t):
            cp = pltpu.make_async_copy(
                w_ref.at[:, pl.ds(((my + offs[t]) % N_DEV) * n_per, n_per)],
                w_buf.at[t % 2],
                w_sems.at[t % 2],
            )
            cp.start()
            return cp

        out_dmas = []

        def flush(row):
            if OUT_HBM:
                cp = pltpu.make_async_copy(
                    ov_ref.at[pl.ds(row * m_per, m_per), :],
                    out_ref.at[pl.ds(row * m_per, m_per), :],
                    ov_sems.at[len(out_dmas)],
                )
                cp.start()
                out_dmas.append(cp)

        descs = []
        amax = jnp.float32(0.0)
        blkmax = {}
        k_half = k // 2
        for t in range(N_DEV):
            if t == 0:
                col0 = ((my + offs[0]) % N_DEV) * n_per
                cp_a = pltpu.make_async_copy(
                    w_ref.at[pl.ds(0, k_half), pl.ds(col0, n_per)],
                    w_buf.at[0, pl.ds(0, k_half)],
                    w_sems.at[0],
                )
                cp_a.start()
                cp_a.wait()
                cp_b = pltpu.make_async_copy(
                    w_ref.at[pl.ds(k_half, k_half), pl.ds(col0, n_per)],
                    w_buf.at[0, pl.ds(k_half, k_half)],
                    w_sems.at[0],
                )
                cp_b.start()
                blk = jnp.dot(
                    x_ref[:, pl.ds(0, k_half)],
                    w_buf[0, pl.ds(0, k_half)],
                    preferred_element_type=jnp.float32,
                )
                cp_b.wait()
                fetch_next = w_fetch(1)
                blk = blk + jnp.dot(
                    x_ref[:, pl.ds(k_half, k_half)],
                    w_buf[0, pl.ds(k_half, k_half)],
                    preferred_element_type=jnp.float32,
                )
            else:
                fetch_next.wait()
                if t + 1 < N_DEV:
                    fetch_next = w_fetch(t + 1)
                blk = jnp.dot(
                    x_ref[...], w_buf[t % 2],
                    preferred_element_type=jnp.float32,
                )
            if t + 1 < N_DEV:
                d = offs[t]
                bm = jnp.maximum(jnp.max(jnp.abs(blk)), jnp.float32(1e-30))
                blkmax[d] = bm
                amax = jnp.maximum(amax, bm)
                iout_ref[d - 1] = jnp.round(blk * (I16MAX / bm)).astype(
                    jnp.int16
                )
                rdma = pltpu.make_async_remote_copy(
                    src_ref=iout_ref.at[d - 1],
                    dst_ref=iin_ref.at[d - 1],
                    send_sem=send_sems.at[d - 1],
                    recv_sem=recv_sems.at[d - 1],
                    device_id=((my + d) % N_DEV,),
                    device_id_type=pl.DeviceIdType.MESH,
                )
                rdma.start()
                descs.append(rdma)
            else:
                amax = jnp.maximum(amax, jnp.max(jnp.abs(blk)))
                ov[pl.ds(my * m_per, m_per), :] = blk

        col = lax.broadcasted_iota(jnp.int32, amax_send.shape, 1)
        msg = jnp.full(amax_send.shape, amax, jnp.float32)
        for d in range(1, N_DEV):
            msg = jnp.where(col == d, blkmax[d], msg)
        amax_send[...] = msg
        am_descs = []
        for d in range(1, N_DEV):
            rdma = pltpu.make_async_remote_copy(
                src_ref=amax_send,
                dst_ref=amax_recv.at[d - 1],
                send_sem=am_send_sems.at[d - 1],
                recv_sem=am_recv_sems.at[d - 1],
                device_id=((my + d) % N_DEV,),
                device_id_type=pl.DeviceIdType.MESH,
            )
            rdma.start()
            am_descs.append(rdma)
        for am in am_descs:
            am.wait_recv()

        gmax = jnp.maximum(amax, jnp.max(amax_recv[...]))
        scale = gmax / 448.0
        inv = 448.0 / gmax

        own = ov[pl.ds(my * m_per, m_per), :]
        ov[pl.ds(my * m_per, m_per), :] = (
            (own * inv).astype(jnp.float8_e4m3fn).astype(jnp.float32) * scale
        )
        flush(my)

        for t in range(N_DEV - 1):
            d = offs[t]
            descs[t].wait_recv()
            src = (my - d) % N_DEV
            bm = jnp.max(amax_recv[d - 1, :, d])
            val = iin_ref[d - 1].astype(jnp.float32) * (bm / I16MAX)
            ov[pl.ds(src * m_per, m_per), :] = (
                (val * inv).astype(jnp.float8_e4m3fn).astype(jnp.float32)
                * scale
            )
            flush(src)

        for cp in out_dmas:
            cp.wait()
        for rdma in descs + am_descs:
            rdma.wait_send()

    out_space = (
        pltpu.MemorySpace.HBM if OUT_HBM else pltpu.MemorySpace.VMEM
    )
    grid_spec = pltpu.PrefetchScalarGridSpec(
        num_scalar_prefetch=0,
        in_specs=[
            pl.BlockSpec(memory_space=pltpu.MemorySpace.VMEM),
            pl.BlockSpec(memory_space=pltpu.MemorySpace.HBM),
        ],
        out_specs=pl.BlockSpec(memory_space=out_space),
        scratch_shapes=[
            pltpu.VMEM(
                (N_DEV * m_per if OUT_HBM else 8, n_per if OUT_HBM else 128),
                jnp.float32,
            ),
            pltpu.VMEM((2, k, n_per), jnp.float32),
            pltpu.VMEM((N_DEV - 1, m_per, n_per), jnp.int16),
            pltpu.VMEM((N_DEV - 1, m_per, n_per), jnp.int16),
            pltpu.VMEM((8, 128), jnp.float32),
            pltpu.VMEM((N_DEV - 1, 8, 128), jnp.float32),
            pltpu.SemaphoreType.DMA((2,)),
            pltpu.SemaphoreType.DMA((N_DEV,)),
            pltpu.SemaphoreType.DMA((N_DEV - 1,)),
            pltpu.SemaphoreType.DMA((N_DEV - 1,)),
            pltpu.SemaphoreType.DMA((N_DEV - 1,)),
            pltpu.SemaphoreType.DMA((N_DEV - 1,)),
        ],
    )
    return pl.pallas_call(
        body,
        out_shape=jax.ShapeDtypeStruct((N_DEV * m_per, n_per), jnp.float32),
        grid_spec=grid_spec,
        compiler_params=pltpu.CompilerParams(
            collective_id=0, vmem_limit_bytes=100 * 1024 * 1024
        ),
    )(x, w_mat)


# device time: 53551 ns/iter; 1.0188x vs baseline; 1.0188x over previous
import jax
import jax.numpy as jnp
from jax import lax
from jax.experimental import pallas as pl
from jax.experimental.pallas import tpu as pltpu

N_DEV = 4
I16MAX = 32767.0
FLOWS = ((2, 0, 512), (1, 0, 512), (3, 0, 256), (3, 256, 256))


def kernel(x, w_mat):
    m_per, k = x.shape
    _, n = w_mat.shape
    n_per = n // N_DEV

    def body(x_ref, w_ref, out_ref, ov_ref, w_buf, iout_ref, iin_ref,
             scl_send, scl_recv, amax_send, amax_recv, w_sems, ov_sems,
             send_sems, recv_sems, scl_send_sems, scl_recv_sems,
             am_send_sems, am_recv_sems):
        my = lax.axis_index("i")

        barrier_sem = pltpu.get_barrier_semaphore()
        for d in range(1, N_DEV):
            pl.semaphore_signal(
                barrier_sem, inc=1,
                device_id=((my + d) % N_DEV,),
                device_id_type=pl.DeviceIdType.MESH,
            )
        pl.semaphore_wait(barrier_sem, N_DEV - 1)

        fetch_offs = (2, 1, 3, 0)

        def w_fetch(t):
            cp = pltpu.make_async_copy(
                w_ref.at[
                    :, pl.ds(((my + fetch_offs[t]) % N_DEV) * n_per, n_per)
                ],
                w_buf.at[t % 2],
                w_sems.at[t % 2],
            )
            cp.start()
            return cp

        def dot(t, co, w_):
            return jnp.dot(
                x_ref[...],
                w_buf[t % 2, :, pl.ds(co, w_)] if w_ < n_per else w_buf[t % 2],
                preferred_element_type=jnp.float32,
            )

        descs = []
        amax = jnp.float32(0.0)
        fetch_cur = w_fetch(0)

        def ship(f, blk):
            d, co, w_ = FLOWS[f]
            bm = jnp.maximum(jnp.max(jnp.abs(blk)), jnp.float32(1e-30))
            iout_ref[d - 1, :, pl.ds(co, w_)] = jnp.round(
                blk * (I16MAX / bm)
            ).astype(jnp.int16)
            rdma = pltpu.make_async_remote_copy(
                src_ref=iout_ref.at[d - 1, :, pl.ds(co, w_)],
                dst_ref=iin_ref.at[d - 1, :, pl.ds(co, w_)],
                send_sem=send_sems.at[f],
                recv_sem=recv_sems.at[f],
                device_id=((my + d) % N_DEV,),
                device_id_type=pl.DeviceIdType.MESH,
            )
            rdma.start()
            scl_send[f] = jnp.full(scl_send.shape[1:], bm, jnp.float32)
            srdma = pltpu.make_async_remote_copy(
                src_ref=scl_send.at[f],
                dst_ref=scl_recv.at[f],
                send_sem=scl_send_sems.at[f],
                recv_sem=scl_recv_sems.at[f],
                device_id=((my + d) % N_DEV,),
                device_id_type=pl.DeviceIdType.MESH,
            )
            srdma.start()
            descs.append(rdma)
            descs.append(srdma)
            return bm

        fetch_cur.wait()
        fetch_cur = w_fetch(1)
        amax = jnp.maximum(amax, ship(0, dot(0, 0, 512)))
        fetch_cur.wait()
        fetch_cur = w_fetch(2)
        amax = jnp.maximum(amax, ship(1, dot(1, 0, 512)))
        fetch_cur.wait()
        fetch_cur = w_fetch(3)
        amax = jnp.maximum(amax, ship(2, dot(2, 0, 256)))
        amax = jnp.maximum(amax, ship(3, dot(2, 256, 256)))
        fetch_cur.wait()
        blk = dot(3, 0, 512)
        amax = jnp.maximum(amax, jnp.max(jnp.abs(blk)))
        ov_ref[pl.ds(my * m_per, m_per), :] = blk

        amax_send[...] = jnp.full(amax_send.shape, amax, jnp.float32)
        am_descs = []
        for d in range(1, N_DEV):
            rdma = pltpu.make_async_remote_copy(
                src_ref=amax_send,
                dst_ref=amax_recv.at[d - 1],
                send_sem=am_send_sems.at[d - 1],
                recv_sem=am_recv_sems.at[d - 1],
                device_id=((my + d) % N_DEV,),
                device_id_type=pl.DeviceIdType.MESH,
            )
            rdma.start()
            am_descs.append(rdma)

        for f, (d, co, w_) in enumerate(FLOWS):
            descs[2 * f].wait_recv()
            descs[2 * f + 1].wait_recv()
            src = (my - d) % N_DEV
            bm = jnp.max(scl_recv[f])
            ov_ref[pl.ds(src * m_per, m_per), pl.ds(co, w_)] = (
                iin_ref[d - 1, :, pl.ds(co, w_)].astype(jnp.float32)
                * (bm / I16MAX)
            )

        for am in am_descs:
            am.wait_recv()
        gmax = jnp.maximum(amax, jnp.max(amax_recv[...]))
        scale = gmax / 448.0
        inv = 448.0 / gmax

        out_dmas = []
        for r in range(N_DEV):
            rows = pl.ds(((my + r) % N_DEV) * m_per, m_per)
            ov_ref[rows, :] = (
                (ov_ref[rows, :] * inv).astype(jnp.float8_e4m3fn)
                .astype(jnp.float32) * scale
            )
            cp = pltpu.make_async_copy(
                ov_ref.at[rows, :], out_ref.at[rows, :], ov_sems.at[r]
            )
            cp.start()
            out_dmas.append(cp)

        for cp in out_dmas:
            cp.wait()
        for rdma in descs + am_descs:
            rdma.wait_send()

    grid_spec = pltpu.PrefetchScalarGridSpec(
        num_scalar_prefetch=0,
        in_specs=[
            pl.BlockSpec(memory_space=pltpu.MemorySpace.VMEM),
            pl.BlockSpec(memory_space=pltpu.MemorySpace.HBM),
        ],
        out_specs=pl.BlockSpec(memory_space=pltpu.MemorySpace.HBM),
        scratch_shapes=[
            pltpu.VMEM((N_DEV * m_per, n_per), jnp.float32),
            pltpu.VMEM((2, k, n_per), jnp.float32),
            pltpu.VMEM((N_DEV - 1, m_per, n_per), jnp.int16),
            pltpu.VMEM((N_DEV - 1, m_per, n_per), jnp.int16),
            pltpu.VMEM((4, 8, 128), jnp.float32),
            pltpu.VMEM((4, 8, 128), jnp.float32),
            pltpu.VMEM((8, 128), jnp.float32),
            pltpu.VMEM((N_DEV - 1, 8, 128), jnp.float32),
            pltpu.SemaphoreType.DMA((2,)),
            pltpu.SemaphoreType.DMA((N_DEV,)),
            pltpu.SemaphoreType.DMA((4,)),
            pltpu.SemaphoreType.DMA((4,)),
            pltpu.SemaphoreType.DMA((4,)),
            pltpu.SemaphoreType.DMA((4,)),
            pltpu.SemaphoreType.DMA((N_DEV - 1,)),
            pltpu.SemaphoreType.DMA((N_DEV - 1,)),
        ],
    )
    return pl.pallas_call(
        body,
        out_shape=jax.ShapeDtypeStruct((N_DEV * m_per, n_per), jnp.float32),
        grid_spec=grid_spec,
        compiler_params=pltpu.CompilerParams(
            collective_id=0, vmem_limit_bytes=100 * 1024 * 1024
        ),
    )(x, w_mat)
